# baseline (speedup 1.0000x reference)
import functools

import jax
import jax.numpy as jnp
from jax import lax
from jax.experimental import pallas as pl
from jax.experimental.pallas import tpu as pltpu
from jax.experimental.pallas import tpu_sc as plsc

NC = 2
NS = 16
NW = NC * NS
L = 16
R = 8
NQ = 4


@functools.partial(jax.jit, static_argnums=(2, 3))
def _permute_rows(x2, perm, n_rows, d):
    rows_per_w = n_rows // NW
    n_chunks = rows_per_w // R
    dq = d // NQ
    mesh = plsc.VectorSubcoreMesh(core_axis_name="c", subcore_axis_name="s")

    def body(x_hbm, perm_hbm, out_hbm, perm_v,
             in0, in1, oq0, oq1, oq2, oq3, si0, si1, so0, so1, so2, so3):
        wid = lax.axis_index("s") * NC + lax.axis_index("c")
        base_r = wid * rows_per_w
        ins, isems = (in0, in1), (si0, si1)
        oqs, osems = (oq0, oq1, oq2, oq3), (so0, so1, so2, so3)

        pltpu.sync_copy(perm_hbm, perm_v)

        def start_in(c, b):
            pltpu.async_copy(
                x_hbm.at[pl.ds(base_r + c * R, R), :], ins[b], isems[b])

        def wait_in(b):
            pltpu.make_async_copy(
                x_hbm.at[pl.ds(0, R), :], ins[b], isems[b]).wait()

        def start_out(c, q):
            pltpu.async_copy(
                oqs[q],
                out_hbm.at[pl.ds(base_r + c * R, R), pl.ds(q * dq, dq)],
                osems[q])

        def wait_out(q):
            pltpu.make_async_copy(
                oqs[q], out_hbm.at[pl.ds(0, R), pl.ds(0, dq)], osems[q]).wait()

        def gather_chunk(c, b, first):
            iv = ins[b]
            wait_in(b)
            for q in range(NQ):
                if not first:
                    wait_out(q)
                ov = oqs[q]

                @plsc.parallel_loop(q * (dq // L), (q + 1) * (dq // L),
                                    unroll=2)
                def jloop(j):
                    idx = perm_v[pl.ds(j * L, L)]
                    jq = j - q * (dq // L)
                    for r in range(R):
                        rvec = jnp.full((L,), r, dtype=jnp.int32)
                        ov[r, pl.ds(jq * L, L)] = plsc.load_gather(
                            iv, [rvec, idx])

                start_out(c, q)

        start_in(0, 0)
        start_in(1, 1)
        gather_chunk(0, 0, True)
        start_in(2, 0)
        gather_chunk(1, 1, False)
        start_in(3, 1)

        @pl.loop(2, n_chunks - 2, step=2)
        def main(g):
            for b in range(2):
                gather_chunk(g + b, b, False)
                start_in(g + b + 2, b)

        gather_chunk(n_chunks - 2, 0, False)
        gather_chunk(n_chunks - 1, 1, False)
        for q in range(NQ):
            wait_out(q)

    fn = pl.kernel(
        body,
        out_type=jax.ShapeDtypeStruct((n_rows, d), jnp.float32),
        mesh=mesh,
        scratch_types=[
            pltpu.VMEM((d,), jnp.int32),
            pltpu.VMEM((R, d), jnp.float32),
            pltpu.VMEM((R, d), jnp.float32),
            pltpu.VMEM((R, dq), jnp.float32),
            pltpu.VMEM((R, dq), jnp.float32),
            pltpu.VMEM((R, dq), jnp.float32),
            pltpu.VMEM((R, dq), jnp.float32),
            pltpu.SemaphoreType.DMA,
            pltpu.SemaphoreType.DMA,
            pltpu.SemaphoreType.DMA,
            pltpu.SemaphoreType.DMA,
            pltpu.SemaphoreType.DMA,
            pltpu.SemaphoreType.DMA,
        ],
        compiler_params=pltpu.CompilerParams(needs_layout_passes=False),
    )
    return fn(x2, perm)


def kernel(x, perm):
    b, s, d = x.shape
    x2 = x.reshape(b * s, d)
    out = _permute_rows(x2, perm, b * s, d)
    return out.reshape(b, s, d)


# speedup vs baseline: 1.0066x; 1.0066x over previous
import functools

import jax
import jax.numpy as jnp
from jax import lax
from jax.experimental import pallas as pl
from jax.experimental.pallas import tpu as pltpu
from jax.experimental.pallas import tpu_sc as plsc

NC = 2
NS = 16
NW = NC * NS
L = 16
R = 8
NQ = 4


@functools.partial(jax.jit, static_argnums=(2, 3))
def _permute_rows(x2, perm, n_rows, d):
    rows_per_w = n_rows // NW
    n_chunks = rows_per_w // R
    dq = d // NQ
    mesh = plsc.VectorSubcoreMesh(core_axis_name="c", subcore_axis_name="s")

    def body(x_hbm, perm_hbm, out_hbm, perm_v,
             in0, in1, oq0, oq1, oq2, oq3, si0, si1, so0, so1, so2, so3):
        wid = lax.axis_index("s") * NC + lax.axis_index("c")
        base_r = wid * rows_per_w
        ins, isems = (in0, in1), (si0, si1)
        oqs, osems = (oq0, oq1, oq2, oq3), (so0, so1, so2, so3)

        def start_in(c, b):
            pltpu.async_copy(
                x_hbm.at[pl.ds(base_r + c * R, R), :], ins[b], isems[b])

        def wait_in(b):
            pltpu.make_async_copy(
                x_hbm.at[pl.ds(0, R), :], ins[b], isems[b]).wait()

        def start_out(c, q):
            pltpu.async_copy(
                oqs[q],
                out_hbm.at[pl.ds(base_r + c * R, R), pl.ds(q * dq, dq)],
                osems[q])

        def wait_out(q):
            pltpu.make_async_copy(
                oqs[q], out_hbm.at[pl.ds(0, R), pl.ds(0, dq)], osems[q]).wait()

        def gather_chunk(c, b, first):
            iv = ins[b]
            wait_in(b)
            for q in range(NQ):
                if not first:
                    wait_out(q)
                ov = oqs[q]

                @plsc.parallel_loop(q * (dq // L), (q + 1) * (dq // L),
                                    unroll=2)
                def jloop(j):
                    idx = perm_v[pl.ds(j * L, L)]
                    jq = j - q * (dq // L)
                    for r in range(R):
                        rvec = jnp.full((L,), r, dtype=jnp.int32)
                        ov[r, pl.ds(jq * L, L)] = plsc.load_gather(
                            iv, [rvec, idx])

                start_out(c, q)

        start_in(0, 0)
        start_in(1, 1)
        pltpu.sync_copy(perm_hbm, perm_v)
        gather_chunk(0, 0, True)
        start_in(2, 0)
        gather_chunk(1, 1, False)
        start_in(3, 1)

        @pl.loop(2, n_chunks - 2, step=2)
        def main(g):
            for b in range(2):
                gather_chunk(g + b, b, False)
                start_in(g + b + 2, b)

        gather_chunk(n_chunks - 2, 0, False)
        gather_chunk(n_chunks - 1, 1, False)
        for q in range(NQ):
            wait_out(q)

    fn = pl.kernel(
        body,
        out_type=jax.ShapeDtypeStruct((n_rows, d), jnp.float32),
        mesh=mesh,
        scratch_types=[
            pltpu.VMEM((d,), jnp.int32),
            pltpu.VMEM((R, d), jnp.float32),
            pltpu.VMEM((R, d), jnp.float32),
            pltpu.VMEM((R, dq), jnp.float32),
            pltpu.VMEM((R, dq), jnp.float32),
            pltpu.VMEM((R, dq), jnp.float32),
            pltpu.VMEM((R, dq), jnp.float32),
            pltpu.SemaphoreType.DMA,
            pltpu.SemaphoreType.DMA,
            pltpu.SemaphoreType.DMA,
            pltpu.SemaphoreType.DMA,
            pltpu.SemaphoreType.DMA,
            pltpu.SemaphoreType.DMA,
        ],
        compiler_params=pltpu.CompilerParams(needs_layout_passes=False),
    )
    return fn(x2, perm)


def kernel(x, perm):
    b, s, d = x.shape
    x2 = x.reshape(b * s, d)
    out = _permute_rows(x2, perm, b * s, d)
    return out.reshape(b, s, d)
